# Initial kernel scaffold; baseline (speedup 1.0000x reference)
import jax
import jax.numpy as jnp
from jax.experimental import pallas as pl


def kernel(node_repr, edge_repr, node_idx_d1, edge_idx_d1, node_idx_d2, edge_idx_d2, node_idx_d4, edge_idx_d4, node_idx_d8, edge_idx_d8, W_self, W_d1, W_d2, W_d4, W_d8, bias):
    raise NotImplementedError("write your pallas kernel here")


# speedup vs baseline: 2.5302x; 2.5302x over previous
import functools

import jax
import jax.numpy as jnp
from jax import lax
from jax.experimental import pallas as pl
from jax.experimental.pallas import tpu as pltpu
from jax.experimental.pallas import tpu_sc as plsc

N = 100000
NODE = 128
EDGE = 16
OUT = 128
E_TOT = 400000
DEGREES = (1, 2, 4, 8)
NPER = 25000

NW = 32
CHW = 784
LO_MAX = NPER - CHW
CH_D = {1: 392, 2: 112, 4: 112, 8: 56}
GMAX = 448


def _gather_slices(total):
    out = []
    off = 0
    while off < total:
        sz = min(128, total - off)
        out.append((off, sz))
        off += sz
    return out


def _sc_gather_sum(node_hbm, edge_hbm,
                   ni1, ei1, ni2, ei2, ni4, ei4, ni8, ei8,
                   sn_hbm, se_hbm,
                   idxn_v, idxe_v, g_node, acc_node, g_edge, acc_edge,
                   sem_n, sem_e):
    nidx = {1: ni1, 2: ni2, 4: ni4, 8: ni8}
    eidx = {1: ei1, 2: ei2, 4: ei4, 8: ei8}
    wid = lax.axis_index("s") * 2 + lax.axis_index("c")
    lo = jnp.minimum(wid * CHW, LO_MAX)

    for di, d in enumerate(DEGREES):
        ch = CH_D[d]
        g = ch * d
        out_off = di * NPER
        n_dst = acc_node if d == 1 else g_node
        e_dst = acc_edge if d == 1 else g_edge
        for k in range(CHW // ch):
            base = lo + k * ch
            pltpu.sync_copy(nidx[d].at[pl.ds(base * d, g)], idxn_v.at[pl.ds(0, g)])
            pltpu.sync_copy(eidx[d].at[pl.ds(base * d, g)], idxe_v.at[pl.ds(0, g)])
            copies = []
            for off, sz in _gather_slices(g):
                copies.append(pltpu.async_copy(
                    node_hbm.at[idxn_v.at[pl.ds(off, sz)]],
                    n_dst.at[pl.ds(off, sz)], sem_n))
                copies.append(pltpu.async_copy(
                    edge_hbm.at[idxe_v.at[pl.ds(off, sz)]],
                    e_dst.at[pl.ds(off, sz)], sem_e))
            for c in copies:
                c.wait()

            if d > 1:
                def body(b, _):
                    row = b * d
                    for cseg in range(NODE // 16):
                        cs = pl.ds(cseg * 16, 16)
                        v = g_node[row, cs]
                        for j in range(1, d):
                            v = v + g_node[row + j, cs]
                        acc_node[b, cs] = v
                    ev = g_edge[row, :]
                    for j in range(1, d):
                        ev = ev + g_edge[row + j, :]
                    acc_edge[b, :] = ev
                    return 0
                lax.fori_loop(0, ch, body, 0)

            pltpu.sync_copy(acc_node.at[pl.ds(0, ch)],
                            sn_hbm.at[pl.ds(out_off + base, ch)])
            pltpu.sync_copy(acc_edge.at[pl.ds(0, ch)],
                            se_hbm.at[pl.ds(out_off + base, ch)])


def _run_sc_gather(node_repr, edge_repr, flat_idx):
    mesh = plsc.VectorSubcoreMesh(core_axis_name="c", subcore_axis_name="s")
    fn = functools.partial(
        pl.kernel,
        out_type=[
            jax.ShapeDtypeStruct((N, NODE), jnp.float32),
            jax.ShapeDtypeStruct((N, EDGE), jnp.float32),
        ],
        mesh=mesh,
        scratch_types=[
            pltpu.VMEM((GMAX,), jnp.int32),
            pltpu.VMEM((GMAX,), jnp.int32),
            pltpu.VMEM((GMAX, NODE), jnp.float32),
            pltpu.VMEM((CH_D[1], NODE), jnp.float32),
            pltpu.VMEM((GMAX, EDGE), jnp.float32),
            pltpu.VMEM((CH_D[1], EDGE), jnp.float32),
            pltpu.SemaphoreType.DMA,
            pltpu.SemaphoreType.DMA,
        ],
        compiler_params=pltpu.CompilerParams(use_tc_tiling_on_sc=False),
    )(_sc_gather_sum)
    return fn(node_repr, edge_repr, *flat_idx)


B_TC = 1000


def _dense_body(node_ref, sn_ref, se_ref, ws_ref, wn_ref, we_ref, bias_ref,
                y_ref, stats_ref):
    i = pl.program_id(0)
    y = jnp.dot(node_ref[...], ws_ref[...], preferred_element_type=jnp.float32)
    y += jnp.dot(sn_ref[...], wn_ref[0], preferred_element_type=jnp.float32)
    y += jnp.dot(se_ref[...], we_ref[0], preferred_element_type=jnp.float32)
    y += bias_ref[...]
    y_ref[...] = y

    @pl.when(i == 0)
    def _():
        stats_ref[...] = jnp.zeros_like(stats_ref)

    s1 = jnp.sum(y, axis=0, keepdims=True)
    s2 = jnp.sum(y * y, axis=0, keepdims=True)
    stats_ref[...] += jnp.concatenate([s1, s2], axis=0)


def _norm_body(y_ref, stats_ref, out_ref):
    s = stats_ref[...]
    mean = s[0:1] * (1.0 / N)
    var = s[1:2] * (1.0 / N) - mean * mean
    inv = lax.rsqrt(var + 1e-5)
    out_ref[...] = jnp.maximum((y_ref[...] - mean) * inv, 0.0)


def kernel(node_repr, edge_repr, node_idx_d1, edge_idx_d1, node_idx_d2,
           edge_idx_d2, node_idx_d4, edge_idx_d4, node_idx_d8, edge_idx_d8,
           W_self, W_d1, W_d2, W_d4, W_d8, bias):
    flat_idx = []
    for ni, ei in ((node_idx_d1, edge_idx_d1), (node_idx_d2, edge_idx_d2),
                   (node_idx_d4, edge_idx_d4), (node_idx_d8, edge_idx_d8)):
        flat_idx.append(ni.reshape(-1))
        flat_idx.append(ei.reshape(-1))

    sn, se = _run_sc_gather(node_repr, edge_repr, flat_idx)

    wn = jnp.stack([W_d1[:NODE], W_d2[:NODE], W_d4[:NODE], W_d8[:NODE]])
    we = jnp.stack([W_d1[NODE:], W_d2[NODE:], W_d4[NODE:], W_d8[NODE:]])

    nblocks = N // B_TC
    per_deg = NPER // B_TC
    y, stats = pl.pallas_call(
        _dense_body,
        grid=(nblocks,),
        in_specs=[
            pl.BlockSpec((B_TC, NODE), lambda i: (i, 0)),
            pl.BlockSpec((B_TC, NODE), lambda i: (i, 0)),
            pl.BlockSpec((B_TC, EDGE), lambda i: (i, 0)),
            pl.BlockSpec((NODE, OUT), lambda i: (0, 0)),
            pl.BlockSpec((1, NODE, OUT), lambda i: (i // per_deg, 0, 0)),
            pl.BlockSpec((1, EDGE, OUT), lambda i: (i // per_deg, 0, 0)),
            pl.BlockSpec((1, OUT), lambda i: (0, 0)),
        ],
        out_specs=[
            pl.BlockSpec((B_TC, OUT), lambda i: (i, 0)),
            pl.BlockSpec((2, OUT), lambda i: (0, 0)),
        ],
        out_shape=[
            jax.ShapeDtypeStruct((N, OUT), jnp.float32),
            jax.ShapeDtypeStruct((2, OUT), jnp.float32),
        ],
    )(node_repr, sn, se, W_self, wn, we, bias)

    out = pl.pallas_call(
        _norm_body,
        grid=(nblocks,),
        in_specs=[
            pl.BlockSpec((B_TC, OUT), lambda i: (i, 0)),
            pl.BlockSpec((2, OUT), lambda i: (0, 0)),
        ],
        out_specs=pl.BlockSpec((B_TC, OUT), lambda i: (i, 0)),
        out_shape=jax.ShapeDtypeStruct((N, OUT), jnp.float32),
    )(y, stats)
    return out
